# baseline (speedup 1.0000x reference)
import jax
import jax.numpy as jnp
from jax.experimental import pallas as pl
from jax.experimental.pallas import tpu as pltpu

_B, _L, _D, _E, _TOP_K = 4, 2048, 4096, 64, 8
_TL = 256
_NT = _L // _TL


def _router_kernel(x_ref, w_ref, b_ref, noise_ref, out_ref, idx_ref,
                   ycur_ref, yprev_ref):
    noisy_list = []
    for b in range(_B):
        y = yprev_ref[b] + b_ref[...]
        logits = y[:, :_E]
        noise_logits = y[:, _E:]
        noisy_list.append(
            logits + noise_ref[b] * jax.nn.softplus(noise_logits))
    mean = (noisy_list[0] + noisy_list[1] + noisy_list[2] + noisy_list[3]) / _B

    iota = jax.lax.broadcasted_iota(jnp.int32, (_TL, _E), 1)
    work = mean
    mask = jnp.zeros((_TL, _E), dtype=jnp.bool_)
    cols = []
    for _ in range(_TOP_K):
        sel = jnp.argmax(work, axis=1).astype(jnp.int32)[:, None]
        hit = iota == sel
        mask = mask | hit
        work = jnp.where(hit, -jnp.inf, work)
        cols.append(sel)
    idx = jnp.concatenate(cols, axis=1)
    idx_ref[...] = jnp.broadcast_to(idx[None], (_B, _TL, _TOP_K))

    for b in range(_B):
        nb = noisy_list[b]
        m = jnp.max(nb, axis=1, keepdims=True)
        e = jnp.where(mask, jnp.exp(nb - m), 0.0)
        out_ref[b] = e / jnp.sum(e, axis=1, keepdims=True)

    w = w_ref[...]
    for b in range(_B):
        ycur_ref[b] = jax.lax.dot_general(
            x_ref[b], w, (((1,), (1,)), ((), ())),
            preferred_element_type=jnp.float32)

    yprev_ref[...] = ycur_ref[...]


def kernel(mh_output, W_route, b_route, W_noise, b_noise):
    W = jnp.concatenate([W_route, W_noise], axis=0)
    bias = jnp.concatenate([b_route, b_noise]).reshape(1, 2 * _E)
    noise = jax.random.normal(jax.random.key(42), (_B, _L, _E), dtype=jnp.float32)

    grid = (_NT + 1,)
    router_output, indices = pl.pallas_call(
        _router_kernel,
        grid=grid,
        in_specs=[
            pl.BlockSpec((_B, _TL, _D),
                         lambda i: (0, jax.lax.min(i, _NT - 1), 0)),
            pl.BlockSpec((2 * _E, _D), lambda i: (0, 0)),
            pl.BlockSpec((1, 2 * _E), lambda i: (0, 0)),
            pl.BlockSpec((_B, _TL, _E),
                         lambda i: (0, jax.lax.max(i - 1, 0), 0)),
        ],
        out_specs=[
            pl.BlockSpec((_B, _TL, _E),
                         lambda i: (0, jax.lax.max(i - 1, 0), 0)),
            pl.BlockSpec((_B, _TL, _TOP_K),
                         lambda i: (0, jax.lax.max(i - 1, 0), 0)),
        ],
        out_shape=[
            jax.ShapeDtypeStruct((_B, _L, _E), jnp.float32),
            jax.ShapeDtypeStruct((_B, _L, _TOP_K), jnp.int32),
        ],
        scratch_shapes=[
            pltpu.VMEM((_B, _TL, 2 * _E), jnp.float32),
            pltpu.VMEM((_B, _TL, 2 * _E), jnp.float32),
        ],
    )(mh_output, W, bias, noise)

    return router_output, indices


# speedup vs baseline: 1.2382x; 1.2382x over previous
import jax
import jax.numpy as jnp
from jax.experimental import pallas as pl
from jax.experimental.pallas import tpu as pltpu

_B, _L, _D, _E, _TOP_K = 4, 2048, 4096, 64, 8
_TL = 256


def _router_kernel(x_ref, w_ref, b_ref, noise_ref, out_ref, idx_ref):
    w = w_ref[...]
    bias = b_ref[...]
    noisy_list = []
    for b in range(_B):
        yt = jax.lax.dot_general(
            w, x_ref[b], (((1,), (1,)), ((), ())),
            preferred_element_type=jnp.float32) + bias
        logits = yt[:_E, :]
        noise_logits = yt[_E:, :]
        noisy_list.append(
            logits + noise_ref[b] * jax.nn.softplus(noise_logits))
    mean = (noisy_list[0] + noisy_list[1] + noisy_list[2] + noisy_list[3]) / _B

    iota = jax.lax.broadcasted_iota(jnp.int32, (_E, _TL), 0)
    work = mean
    mask = jnp.zeros((_E, _TL), dtype=jnp.bool_)
    cols = []
    for _ in range(_TOP_K):
        m = jnp.max(work, axis=0, keepdims=True)
        sel = jnp.min(jnp.where(work == m, iota, _E), axis=0, keepdims=True)
        hit = iota == sel
        mask = mask | hit
        work = jnp.where(hit, -jnp.inf, work)
        cols.append(sel)
    idx = jnp.concatenate(cols, axis=0)
    idx_ref[...] = jnp.broadcast_to(
        idx.T[None], (_B, _TL, _TOP_K)).astype(jnp.int32)

    for b in range(_B):
        nb = noisy_list[b]
        m = jnp.max(nb, axis=0, keepdims=True)
        e = jnp.where(mask, jnp.exp(nb - m), 0.0)
        out_ref[b] = (e / jnp.sum(e, axis=0, keepdims=True)).T


def kernel(mh_output, W_route, b_route, W_noise, b_noise):
    W = jnp.concatenate([W_route, W_noise], axis=0)
    bias = jnp.concatenate([b_route, b_noise]).reshape(2 * _E, 1)
    noise = jax.random.normal(jax.random.key(42), (_B, _L, _E), dtype=jnp.float32)
    noise_t = jnp.transpose(noise, (0, 2, 1))

    grid = (_L // _TL,)
    router_output, indices = pl.pallas_call(
        _router_kernel,
        grid=grid,
        in_specs=[
            pl.BlockSpec((_B, _TL, _D), lambda i: (0, i, 0)),
            pl.BlockSpec((2 * _E, _D), lambda i: (0, 0)),
            pl.BlockSpec((2 * _E, 1), lambda i: (0, 0)),
            pl.BlockSpec((_B, _E, _TL), lambda i: (0, 0, i)),
        ],
        out_specs=[
            pl.BlockSpec((_B, _TL, _E), lambda i: (0, i, 0)),
            pl.BlockSpec((_B, _TL, _TOP_K), lambda i: (0, i, 0)),
        ],
        out_shape=[
            jax.ShapeDtypeStruct((_B, _L, _E), jnp.float32),
            jax.ShapeDtypeStruct((_B, _L, _TOP_K), jnp.int32),
        ],
        compiler_params=pltpu.CompilerParams(
            dimension_semantics=("parallel",)),
    )(mh_output, W, bias, noise_t)

    return router_output, indices


# speedup vs baseline: 1.2392x; 1.0008x over previous
import jax
import jax.numpy as jnp
from jax.experimental import pallas as pl
from jax.experimental.pallas import tpu as pltpu

_B, _L, _D, _E, _TOP_K = 4, 2048, 4096, 64, 8
_TL = 256
_H = _D // 2


def _router_kernel(x1_ref, x2_ref, w_ref, b_ref, noise_ref, out_ref, idx_ref):
    w = w_ref[...]
    x1 = x1_ref[...].reshape(_B * _TL, _H)
    x2 = x2_ref[...].reshape(_B * _TL, _H)
    yt = (jax.lax.dot_general(
              w[:, :_H], x1, (((1,), (1,)), ((), ())),
              preferred_element_type=jnp.float32)
          + jax.lax.dot_general(
              w[:, _H:], x2, (((1,), (1,)), ((), ())),
              preferred_element_type=jnp.float32)
          + b_ref[...])
    noisy_list = []
    for b in range(_B):
        sl = slice(b * _TL, (b + 1) * _TL)
        noisy_list.append(
            yt[:_E, sl]
            + noise_ref[b] * jax.nn.softplus(yt[_E:, sl]))
    mean = (noisy_list[0] + noisy_list[1] + noisy_list[2] + noisy_list[3]) / _B

    iota = jax.lax.broadcasted_iota(jnp.int32, (_E, _TL), 0)
    work = mean
    mask = jnp.zeros((_E, _TL), dtype=jnp.bool_)
    cols = []
    for _ in range(_TOP_K):
        m = jnp.max(work, axis=0, keepdims=True)
        sel = jnp.min(jnp.where(work == m, iota, _E), axis=0, keepdims=True)
        hit = iota == sel
        mask = mask | hit
        work = jnp.where(hit, -jnp.inf, work)
        cols.append(sel)
    idx = jnp.concatenate(cols, axis=0)
    idx_ref[...] = jnp.broadcast_to(
        idx.T[None], (_B, _TL, _TOP_K)).astype(jnp.int32)

    for b in range(_B):
        nb = noisy_list[b]
        m = jnp.max(nb, axis=0, keepdims=True)
        e = jnp.where(mask, jnp.exp(nb - m), 0.0)
        out_ref[b] = (e / jnp.sum(e, axis=0, keepdims=True)).T


def kernel(mh_output, W_route, b_route, W_noise, b_noise):
    W = jnp.concatenate([W_route, W_noise], axis=0)
    bias = jnp.concatenate([b_route, b_noise]).reshape(2 * _E, 1)
    noise = jax.random.normal(jax.random.key(42), (_B, _L, _E), dtype=jnp.float32)
    noise_t = jnp.transpose(noise, (0, 2, 1))

    grid = (_L // _TL,)
    router_output, indices = pl.pallas_call(
        _router_kernel,
        grid=grid,
        in_specs=[
            pl.BlockSpec((_B, _TL, _H), lambda i: (0, i, 0)),
            pl.BlockSpec((_B, _TL, _H), lambda i: (0, i, 1)),
            pl.BlockSpec((2 * _E, _D), lambda i: (0, 0)),
            pl.BlockSpec((2 * _E, 1), lambda i: (0, 0)),
            pl.BlockSpec((_B, _E, _TL), lambda i: (0, 0, i)),
        ],
        out_specs=[
            pl.BlockSpec((_B, _TL, _E), lambda i: (0, i, 0)),
            pl.BlockSpec((_B, _TL, _TOP_K), lambda i: (0, i, 0)),
        ],
        out_shape=[
            jax.ShapeDtypeStruct((_B, _L, _E), jnp.float32),
            jax.ShapeDtypeStruct((_B, _L, _TOP_K), jnp.int32),
        ],
        compiler_params=pltpu.CompilerParams(
            dimension_semantics=("parallel",)),
    )(mh_output, mh_output, W, bias, noise_t)

    return router_output, indices


# speedup vs baseline: 1.2952x; 1.0452x over previous
import jax
import jax.numpy as jnp
from jax.experimental import pallas as pl
from jax.experimental.pallas import tpu as pltpu

_B, _L, _D, _E, _TOP_K = 4, 2048, 4096, 64, 8
_TL = 256
_H = _D // 2


def _router_kernel(x1_ref, x2_ref, w_ref, b_ref, noise_ref, out_ref, idx_ref):
    w = w_ref[...]
    x1 = x1_ref[...].reshape(_B * _TL, _H)
    x2 = x2_ref[...].reshape(_B * _TL, _H)
    yt = (jax.lax.dot_general(
              w[:, :_H], x1, (((1,), (1,)), ((), ())),
              preferred_element_type=jnp.float32)
          + jax.lax.dot_general(
              w[:, _H:], x2, (((1,), (1,)), ((), ())),
              preferred_element_type=jnp.float32)
          + b_ref[...])
    noisy_list = []
    for b in range(_B):
        sl = slice(b * _TL, (b + 1) * _TL)
        noisy_list.append(
            yt[:_E, sl]
            + noise_ref[b] * jax.nn.softplus(yt[_E:, sl]))
    mean = (noisy_list[0] + noisy_list[1] + noisy_list[2] + noisy_list[3]) / _B

    iota = jax.lax.broadcasted_iota(jnp.int32, (_E, _TL), 0)
    work = mean
    mask = jnp.zeros((_E, _TL), dtype=jnp.bool_)
    cols = []
    for _ in range(_TOP_K):
        m = jnp.max(work, axis=0, keepdims=True)
        sel = jnp.min(jnp.where(work == m, iota, _E), axis=0, keepdims=True)
        hit = iota == sel
        mask = mask | hit
        work = jnp.where(hit, -jnp.inf, work)
        cols.append(sel)
    idx = jnp.concatenate(cols, axis=0)
    idx_ref[...] = jnp.broadcast_to(
        idx.T[None], (_B, _TL, _TOP_K)).astype(jnp.int32)

    for b in range(_B):
        nb = noisy_list[b]
        m = jnp.max(nb, axis=0, keepdims=True)
        e = jnp.where(mask, jnp.exp(nb - m), 0.0)
        out_ref[b] = e / jnp.sum(e, axis=0, keepdims=True)


def kernel(mh_output, W_route, b_route, W_noise, b_noise):
    W = jnp.concatenate([W_route, W_noise], axis=0)
    bias = jnp.concatenate([b_route, b_noise]).reshape(2 * _E, 1)
    noise = jax.random.normal(jax.random.key(42), (_B, _L, _E), dtype=jnp.float32)
    noise_t = jnp.transpose(noise, (0, 2, 1))

    grid = (_L // _TL,)
    router_output, indices = pl.pallas_call(
        _router_kernel,
        grid=grid,
        in_specs=[
            pl.BlockSpec((_B, _TL, _H), lambda i: (0, i, 0)),
            pl.BlockSpec((_B, _TL, _H), lambda i: (0, i, 1)),
            pl.BlockSpec((2 * _E, _D), lambda i: (0, 0)),
            pl.BlockSpec((2 * _E, 1), lambda i: (0, 0)),
            pl.BlockSpec((_B, _E, _TL), lambda i: (0, 0, i)),
        ],
        out_specs=[
            pl.BlockSpec((_B, _E, _TL), lambda i: (0, 0, i)),
            pl.BlockSpec((_B, _TL, _TOP_K), lambda i: (0, i, 0)),
        ],
        out_shape=[
            jax.ShapeDtypeStruct((_B, _E, _L), jnp.float32),
            jax.ShapeDtypeStruct((_B, _L, _TOP_K), jnp.int32),
        ],
        compiler_params=pltpu.CompilerParams(
            dimension_semantics=("parallel",)),
    )(mh_output, mh_output, W, bias, noise_t)

    return jnp.transpose(router_output, (0, 2, 1)), indices
